# Initial kernel scaffold; baseline (speedup 1.0000x reference)
import jax
import jax.numpy as jnp
from jax.experimental import pallas as pl


def kernel(x, edge_index, W1, b1, W2, b2, A1, ab1, A2, ab2, Wc, bc):
    raise NotImplementedError("write your pallas kernel here")


# speedup vs baseline: 4.3546x; 4.3546x over previous
import functools

import jax
import jax.numpy as jnp
from jax import lax
from jax.experimental import pallas as pl
from jax.experimental.pallas import tpu as pltpu
from jax.experimental.pallas import tpu_sc as plsc

N = 10000
E = 320000
D = 128
H = 128

NC = 2
NS = 16
NW = NC * NS
T = E // NW
C = 80
NCH = T // C

_mesh = plsc.VectorSubcoreMesh(core_axis_name="c", subcore_axis_name="s")


def _leaky(v, slope=0.01):
    return jnp.where(v >= 0, v, slope * v)


P = 3
DSTR = 216
RNG = NS * DSTR
ZSTR = 224
ACC_ROWS = NS * ZSTR
TRASH = RNG + 16


@functools.partial(
    pl.kernel,
    out_type=jax.ShapeDtypeStruct((NC, P, NS, DSTR, H), jnp.float32),
    mesh=_mesh,
    scratch_types=[
        pltpu.VMEM((C,), jnp.int32),
        pltpu.VMEM((C,), jnp.int32),
        pltpu.VMEM((C,), jnp.int32),
        pltpu.VMEM((C, H), jnp.float32),
        pltpu.VMEM((ZSTR, H), jnp.float32),
        pltpu.VMEM((DSTR, H), jnp.float32),
        pltpu.SemaphoreType.DMA,
        pltpu.VMEM_SHARED((ACC_ROWS, H), jnp.float32),
    ],
)
def _sc_agg(g, src, dst, zeros, out, srcb, dstb, dstlb, rows, zb, db, sem,
            acc):
    c = lax.axis_index("c")
    s = lax.axis_index("s")
    wid = s * NC + c
    pltpu.sync_copy(zeros, zb)
    for p in range(P):
        lo = p * RNG
        pltpu.sync_copy(zb, acc.at[pl.ds(s * ZSTR, ZSTR)])
        plsc.subcore_barrier()

        def step(j, carry):
            base = wid * T + j * C
            pltpu.sync_copy(src.at[pl.ds(base, C)], srcb)
            pltpu.sync_copy(dst.at[pl.ds(base, C)], dstb)
            for l in range(C // 16):
                dv = dstb[pl.ds(16 * l, 16)]
                inr = (dv >= lo) & (dv < lo + RNG)
                dstlb[pl.ds(16 * l, 16)] = jnp.where(inr, dv - lo, TRASH)
            pltpu.async_copy(g.at[srcb], rows, sem).wait()
            pltpu.sync_copy(rows, acc.at[dstlb], add=True)
            return carry

        lax.fori_loop(0, NCH, step, 0)
        plsc.subcore_barrier()
        pltpu.sync_copy(acc.at[pl.ds(s * DSTR, DSTR)], db)
        pltpu.sync_copy(db, out.at[c, p, s])
        plsc.subcore_barrier()


@functools.partial(
    pl.kernel,
    out_type=jax.ShapeDtypeStruct((NC, P, NS, DSTR, H), jnp.float32),
    mesh=_mesh,
    scratch_types=[
        pltpu.VMEM((C,), jnp.int32),
        pltpu.VMEM((C,), jnp.int32),
        pltpu.VMEM((C, H), jnp.float32),
        pltpu.VMEM((ZSTR, H), jnp.float32),
        pltpu.VMEM((DSTR, H), jnp.float32),
        pltpu.VMEM_SHARED((ACC_ROWS, H), jnp.float32),
    ],
)
def _sc_deg(dst, ones_c, zeros, out, dstb, dstlb, rows, zb, db, acc):
    c = lax.axis_index("c")
    s = lax.axis_index("s")
    wid = s * NC + c
    pltpu.sync_copy(zeros, zb)
    pltpu.sync_copy(ones_c, rows)
    for p in range(P):
        lo = p * RNG
        pltpu.sync_copy(zb, acc.at[pl.ds(s * ZSTR, ZSTR)])
        plsc.subcore_barrier()

        def step(j, carry):
            base = wid * T + j * C
            pltpu.sync_copy(dst.at[pl.ds(base, C)], dstb)
            for l in range(C // 16):
                dv = dstb[pl.ds(16 * l, 16)]
                inr = (dv >= lo) & (dv < lo + RNG)
                dstlb[pl.ds(16 * l, 16)] = jnp.where(inr, dv - lo, TRASH)
            pltpu.sync_copy(rows, acc.at[dstlb], add=True)
            return carry

        lax.fori_loop(0, NCH, step, 0)
        plsc.subcore_barrier()
        pltpu.sync_copy(acc.at[pl.ds(s * DSTR, DSTR)], db)
        pltpu.sync_copy(db, out.at[c, p, s])
        plsc.subcore_barrier()


_BLK = 1000
_GRID = N // _BLK


def _dinv_of(degp):
    deg = degp[0, :, 0] + degp[1, :, 0] + 1.0
    return lax.rsqrt(deg)


def _tc1_body(x_ref, w1_ref, degp_ref, h_ref, g_ref):
    h = jnp.dot(x_ref[...], w1_ref[...], preferred_element_type=jnp.float32)
    dinv = _dinv_of(degp_ref[...])
    h_ref[...] = h
    g_ref[...] = h * dinv[:, None]


def _tc2_body(aggp_ref, hin_ref, degp_ref, w_ref, b_ref, h2_ref, g2_ref):
    dinv = _dinv_of(degp_ref[...])
    agg = aggp_ref[0] + aggp_ref[1]
    t = _leaky(agg * dinv[:, None] + hin_ref[...] * (dinv * dinv)[:, None]
               + b_ref[...])
    h2 = jnp.dot(t, w_ref[...], preferred_element_type=jnp.float32)
    h2_ref[...] = h2
    g2_ref[...] = h2 * dinv[:, None]


def _tc3_body(aggp_ref, hin_ref, degp_ref, b_ref, a1_ref, ab1_ref,
              h_ref, t3_ref):
    dinv = _dinv_of(degp_ref[...])
    agg = aggp_ref[0] + aggp_ref[1]
    h = _leaky(agg * dinv[:, None] + hin_ref[...] * (dinv * dinv)[:, None]
               + b_ref[...])
    h_ref[...] = h
    t3_ref[...] = _leaky(
        jnp.dot(h, a1_ref[...], preferred_element_type=jnp.float32)
        + ab1_ref[...])


def _tc4_body(t3_ref, h_ref, a2_ref, ab2_ref, wc_ref, bc_ref,
              out_ref, a_ref):
    s = jnp.dot(t3_ref[...], a2_ref[...],
                preferred_element_type=jnp.float32) + ab2_ref[0, 0]
    m = jnp.max(s)
    e = jnp.exp(s - m)
    a = e / jnp.sum(e)
    a_ref[...] = a
    z = jnp.sum(h_ref[...] * a, axis=0)
    out_ref[...] = (jnp.sum(z[:, None] * wc_ref[...], axis=0, keepdims=True)
                    + bc_ref[...])


def _row_spec(w):
    return pl.BlockSpec((_BLK, w), lambda i: (i, 0))


def _full_spec(shape):
    return pl.BlockSpec(shape, lambda i: tuple(0 for _ in shape))


_aggp_spec = pl.BlockSpec((NC, _BLK, H), lambda i: (0, i, 0))

_tc1 = pl.pallas_call(
    _tc1_body,
    grid=(_GRID,),
    in_specs=[_row_spec(D), _full_spec((D, H)), _aggp_spec],
    out_specs=[_row_spec(H), _row_spec(H)],
    out_shape=[jax.ShapeDtypeStruct((N, H), jnp.float32)] * 2,
)

_tc2 = pl.pallas_call(
    _tc2_body,
    grid=(_GRID,),
    in_specs=[_aggp_spec, _row_spec(H), _aggp_spec, _full_spec((H, H)),
              _full_spec((1, H))],
    out_specs=[_row_spec(H), _row_spec(H)],
    out_shape=[jax.ShapeDtypeStruct((N, H), jnp.float32)] * 2,
)

_tc3 = pl.pallas_call(
    _tc3_body,
    grid=(_GRID,),
    in_specs=[_aggp_spec, _row_spec(H), _aggp_spec, _full_spec((1, H)),
              _full_spec((H, 128)), _full_spec((1, 128))],
    out_specs=[_row_spec(H), _row_spec(128)],
    out_shape=[jax.ShapeDtypeStruct((N, H), jnp.float32),
               jax.ShapeDtypeStruct((N, 128), jnp.float32)],
)

_tc4 = pl.pallas_call(
    _tc4_body,
    in_specs=[pl.BlockSpec((N, 128), lambda: (0, 0)),
              pl.BlockSpec((N, H), lambda: (0, 0)),
              pl.BlockSpec((128, 1), lambda: (0, 0)),
              pl.BlockSpec((1, 1), lambda: (0, 0)),
              pl.BlockSpec((H, 1), lambda: (0, 0)),
              pl.BlockSpec((1, 1), lambda: (0, 0))],
    out_specs=[pl.BlockSpec((1, 1), lambda: (0, 0)),
               pl.BlockSpec((N, 1), lambda: (0, 0))],
    out_shape=[jax.ShapeDtypeStruct((1, 1), jnp.float32),
               jax.ShapeDtypeStruct((N, 1), jnp.float32)],
)


def kernel(x, edge_index, W1, b1, W2, b2, A1, ab1, A2, ab2, Wc, bc):
    src = edge_index[0]
    dst = edge_index[1]
    zeros128 = jnp.zeros((ZSTR, H), jnp.float32)
    ones_chunk = jnp.ones((C, H), jnp.float32)

    degp = _sc_deg(dst, ones_chunk, zeros128).reshape(NC, P * RNG, H)
    h1, g1 = _tc1(x, W1, degp)
    aggp1 = _sc_agg(g1, src, dst, zeros128).reshape(NC, P * RNG, H)
    h2, g2 = _tc2(aggp1, h1, degp, W2, b1.reshape(1, H))
    aggp2 = _sc_agg(g2, src, dst, zeros128).reshape(NC, P * RNG, H)
    h, t3 = _tc3(aggp2, h2, degp, b2.reshape(1, H), A1, ab1.reshape(1, 128))
    out11, a_col = _tc4(t3, h, A2, ab2.reshape(1, 1), Wc, bc.reshape(1, 1))
    return (out11.reshape(1), a_col.reshape(N), h)


# speedup vs baseline: 6.4837x; 1.4889x over previous
import functools

import jax
import jax.numpy as jnp
from jax import lax
from jax.experimental import pallas as pl
from jax.experimental.pallas import tpu as pltpu
from jax.experimental.pallas import tpu_sc as plsc

N = 10000
E = 320000
D = 128
H = 128

NC = 2
NS = 16
NW = NC * NS
T = E // NW
C = 80
NCH = T // C

_mesh = plsc.VectorSubcoreMesh(core_axis_name="c", subcore_axis_name="s")


def _leaky(v, slope=0.01):
    return jnp.where(v >= 0, v, slope * v)


P = 3
DSTR = 216
RNG = NS * DSTR
ZSTR = 224
ACC_ROWS = NS * ZSTR
TRASH = RNG + 16


NPAIR = (NCH - 1) // 2
assert 2 * NPAIR + 1 == NCH


@functools.partial(
    pl.kernel,
    out_type=jax.ShapeDtypeStruct((NC, P, NS, DSTR, H), jnp.float32),
    mesh=_mesh,
    scratch_types=[
        pltpu.VMEM((C,), jnp.int32),
        pltpu.VMEM((C,), jnp.int32),
        pltpu.VMEM((C,), jnp.int32),
        pltpu.VMEM((C,), jnp.int32),
        pltpu.VMEM((C,), jnp.int32),
        pltpu.VMEM((C, H), jnp.float32),
        pltpu.VMEM((C, H), jnp.float32),
        pltpu.VMEM((ZSTR, H), jnp.float32),
        pltpu.VMEM((DSTR, H), jnp.float32),
        pltpu.SemaphoreType.DMA,
        pltpu.SemaphoreType.DMA,
        pltpu.VMEM_SHARED((ACC_ROWS, H), jnp.float32),
    ],
)
def _sc_agg(g, src, dst, zeros, out, srcb0, srcb1, dstb, dstlb0, dstlb1,
            rows0, rows1, zb, db, sem0, sem1, acc):
    c = lax.axis_index("c")
    s = lax.axis_index("s")
    wid = s * NC + c
    pltpu.sync_copy(zeros, zb)

    def issue(j, srcb, rows, sem):
        pltpu.sync_copy(src.at[pl.ds(wid * T + j * C, C)], srcb)
        pltpu.async_copy(g.at[srcb], rows, sem)

    def transform(j, lo, dstlb):
        pltpu.sync_copy(dst.at[pl.ds(wid * T + j * C, C)], dstb)
        for l in range(C // 16):
            dv = dstb[pl.ds(16 * l, 16)]
            inr = (dv >= lo) & (dv < lo + RNG)
            dstlb[pl.ds(16 * l, 16)] = jnp.where(inr, dv - lo, TRASH)

    def wait(rows, sem):
        pltpu.make_async_copy(g.at[pl.ds(0, C)], rows, sem).wait()

    for p in range(P):
        lo = p * RNG
        pltpu.sync_copy(zb, acc.at[pl.ds(s * ZSTR, ZSTR)])
        plsc.subcore_barrier()

        issue(0, srcb0, rows0, sem0)
        transform(0, lo, dstlb0)

        def pair(i, carry):
            a = 2 * i
            issue(a + 1, srcb1, rows1, sem1)
            wait(rows0, sem0)
            pltpu.sync_copy(rows0, acc.at[dstlb0], add=True)
            issue(a + 2, srcb0, rows0, sem0)
            transform(a + 1, lo, dstlb1)
            wait(rows1, sem1)
            pltpu.sync_copy(rows1, acc.at[dstlb1], add=True)
            transform(a + 2, lo, dstlb0)
            return carry

        lax.fori_loop(0, NPAIR, pair, 0)
        wait(rows0, sem0)
        pltpu.sync_copy(rows0, acc.at[dstlb0], add=True)

        plsc.subcore_barrier()
        pltpu.sync_copy(acc.at[pl.ds(s * DSTR, DSTR)], db)
        pltpu.sync_copy(db, out.at[c, p, s])
        plsc.subcore_barrier()


@functools.partial(
    pl.kernel,
    out_type=jax.ShapeDtypeStruct((NC, P, NS, DSTR, H), jnp.float32),
    mesh=_mesh,
    scratch_types=[
        pltpu.VMEM((C,), jnp.int32),
        pltpu.VMEM((C,), jnp.int32),
        pltpu.VMEM((C, H), jnp.float32),
        pltpu.VMEM((ZSTR, H), jnp.float32),
        pltpu.VMEM((DSTR, H), jnp.float32),
        pltpu.VMEM_SHARED((ACC_ROWS, H), jnp.float32),
    ],
)
def _sc_deg(dst, ones_c, zeros, out, dstb, dstlb, rows, zb, db, acc):
    c = lax.axis_index("c")
    s = lax.axis_index("s")
    wid = s * NC + c
    pltpu.sync_copy(zeros, zb)
    pltpu.sync_copy(ones_c, rows)
    for p in range(P):
        lo = p * RNG
        pltpu.sync_copy(zb, acc.at[pl.ds(s * ZSTR, ZSTR)])
        plsc.subcore_barrier()

        def step(j, carry):
            base = wid * T + j * C
            pltpu.sync_copy(dst.at[pl.ds(base, C)], dstb)
            for l in range(C // 16):
                dv = dstb[pl.ds(16 * l, 16)]
                inr = (dv >= lo) & (dv < lo + RNG)
                dstlb[pl.ds(16 * l, 16)] = jnp.where(inr, dv - lo, TRASH)
            pltpu.sync_copy(rows, acc.at[dstlb], add=True)
            return carry

        lax.fori_loop(0, NCH, step, 0)
        plsc.subcore_barrier()
        pltpu.sync_copy(acc.at[pl.ds(s * DSTR, DSTR)], db)
        pltpu.sync_copy(db, out.at[c, p, s])
        plsc.subcore_barrier()


_BLK = 1000
_GRID = N // _BLK


def _dinv_of(degp):
    deg = degp[0, :, 0] + degp[1, :, 0] + 1.0
    return lax.rsqrt(deg)


def _tc1_body(x_ref, w1_ref, degp_ref, h_ref, g_ref):
    h = jnp.dot(x_ref[...], w1_ref[...], preferred_element_type=jnp.float32)
    dinv = _dinv_of(degp_ref[...])
    h_ref[...] = h
    g_ref[...] = h * dinv[:, None]


def _tc2_body(aggp_ref, hin_ref, degp_ref, w_ref, b_ref, h2_ref, g2_ref):
    dinv = _dinv_of(degp_ref[...])
    agg = aggp_ref[0] + aggp_ref[1]
    t = _leaky(agg * dinv[:, None] + hin_ref[...] * (dinv * dinv)[:, None]
               + b_ref[...])
    h2 = jnp.dot(t, w_ref[...], preferred_element_type=jnp.float32)
    h2_ref[...] = h2
    g2_ref[...] = h2 * dinv[:, None]


def _tc3_body(aggp_ref, hin_ref, degp_ref, b_ref, a1_ref, ab1_ref,
              h_ref, t3_ref):
    dinv = _dinv_of(degp_ref[...])
    agg = aggp_ref[0] + aggp_ref[1]
    h = _leaky(agg * dinv[:, None] + hin_ref[...] * (dinv * dinv)[:, None]
               + b_ref[...])
    h_ref[...] = h
    t3_ref[...] = _leaky(
        jnp.dot(h, a1_ref[...], preferred_element_type=jnp.float32)
        + ab1_ref[...])


def _tc4_body(t3_ref, h_ref, a2_ref, ab2_ref, wc_ref, bc_ref,
              out_ref, a_ref):
    s = jnp.dot(t3_ref[...], a2_ref[...],
                preferred_element_type=jnp.float32) + ab2_ref[0, 0]
    m = jnp.max(s)
    e = jnp.exp(s - m)
    a = e / jnp.sum(e)
    a_ref[...] = a
    z = jnp.sum(h_ref[...] * a, axis=0)
    out_ref[...] = (jnp.sum(z[:, None] * wc_ref[...], axis=0, keepdims=True)
                    + bc_ref[...])


def _row_spec(w):
    return pl.BlockSpec((_BLK, w), lambda i: (i, 0))


def _full_spec(shape):
    return pl.BlockSpec(shape, lambda i: tuple(0 for _ in shape))


_aggp_spec = pl.BlockSpec((NC, _BLK, H), lambda i: (0, i, 0))

_tc1 = pl.pallas_call(
    _tc1_body,
    grid=(_GRID,),
    in_specs=[_row_spec(D), _full_spec((D, H)), _aggp_spec],
    out_specs=[_row_spec(H), _row_spec(H)],
    out_shape=[jax.ShapeDtypeStruct((N, H), jnp.float32)] * 2,
)

_tc2 = pl.pallas_call(
    _tc2_body,
    grid=(_GRID,),
    in_specs=[_aggp_spec, _row_spec(H), _aggp_spec, _full_spec((H, H)),
              _full_spec((1, H))],
    out_specs=[_row_spec(H), _row_spec(H)],
    out_shape=[jax.ShapeDtypeStruct((N, H), jnp.float32)] * 2,
)

_tc3 = pl.pallas_call(
    _tc3_body,
    grid=(_GRID,),
    in_specs=[_aggp_spec, _row_spec(H), _aggp_spec, _full_spec((1, H)),
              _full_spec((H, 128)), _full_spec((1, 128))],
    out_specs=[_row_spec(H), _row_spec(128)],
    out_shape=[jax.ShapeDtypeStruct((N, H), jnp.float32),
               jax.ShapeDtypeStruct((N, 128), jnp.float32)],
)

_tc4 = pl.pallas_call(
    _tc4_body,
    in_specs=[pl.BlockSpec((N, 128), lambda: (0, 0)),
              pl.BlockSpec((N, H), lambda: (0, 0)),
              pl.BlockSpec((128, 1), lambda: (0, 0)),
              pl.BlockSpec((1, 1), lambda: (0, 0)),
              pl.BlockSpec((H, 1), lambda: (0, 0)),
              pl.BlockSpec((1, 1), lambda: (0, 0))],
    out_specs=[pl.BlockSpec((1, 1), lambda: (0, 0)),
               pl.BlockSpec((N, 1), lambda: (0, 0))],
    out_shape=[jax.ShapeDtypeStruct((1, 1), jnp.float32),
               jax.ShapeDtypeStruct((N, 1), jnp.float32)],
)


def kernel(x, edge_index, W1, b1, W2, b2, A1, ab1, A2, ab2, Wc, bc):
    src = edge_index[0]
    dst = edge_index[1]
    zeros128 = jnp.zeros((ZSTR, H), jnp.float32)
    ones_chunk = jnp.ones((C, H), jnp.float32)

    degp = _sc_deg(dst, ones_chunk, zeros128).reshape(NC, P * RNG, H)
    h1, g1 = _tc1(x, W1, degp)
    aggp1 = _sc_agg(g1, src, dst, zeros128).reshape(NC, P * RNG, H)
    h2, g2 = _tc2(aggp1, h1, degp, W2, b1.reshape(1, H))
    aggp2 = _sc_agg(g2, src, dst, zeros128).reshape(NC, P * RNG, H)
    h, t3 = _tc3(aggp2, h2, degp, b2.reshape(1, H), A1, ab1.reshape(1, 128))
    out11, a_col = _tc4(t3, h, A2, ab2.reshape(1, 1), Wc, bc.reshape(1, 1))
    return (out11.reshape(1), a_col.reshape(N), h)
